# baseline (speedup 1.0000x reference)
import functools

import jax
import jax.numpy as jnp
from jax import lax
from jax.experimental import pallas as pl
from jax.experimental.pallas import tpu as pltpu
from jax.experimental.pallas import tpu_sc as plsc

D_MODEL = 1024
HID = 4096
N_EXP = 8
TOPK = 2
TOK = 2048
TILE = 256
MT = TOK // TILE
SLOT = MT * TILE
YS_ROWS = N_EXP * SLOT
HHALF = HID // 2

_SC_INFO = plsc.get_sparse_core_info()
_NC = _SC_INFO.num_cores
_NS = _SC_INFO.num_subcores
NW = _NC * _NS

_MESH = plsc.VectorSubcoreMesh(core_axis_name="c", subcore_axis_name="s")


def _router_body(x_ref, wr_ref, br_ref,
                 p0_ref, p1_ref, w0_ref, w1_ref, nt_ref):
    logits = jnp.dot(x_ref[...], wr_ref[...], preferred_element_type=jnp.float32)
    logits = logits + br_ref[...]
    cols = lax.broadcasted_iota(jnp.int32, logits.shape, 1)
    neg = jnp.float32(-jnp.inf)
    l0 = jnp.where(cols < N_EXP, logits, neg)
    v0 = jnp.max(l0, axis=1, keepdims=True)
    a0 = jnp.min(jnp.where(l0 == v0, cols, N_EXP), axis=1, keepdims=True)
    l1 = jnp.where(cols == a0, neg, l0)
    v1 = jnp.max(l1, axis=1, keepdims=True)
    a1 = jnp.min(jnp.where(l1 == v1, cols, N_EXP), axis=1, keepdims=True)
    t = jnp.exp(v1 - v0)
    w0 = 1.0 / (1.0 + t)
    w0_ref[...] = jnp.transpose(w0)
    w1_ref[...] = jnp.transpose(1.0 - w0)

    a0r = jnp.transpose(a0)
    a1r = jnp.transpose(a1)
    erow = lax.broadcasted_iota(jnp.int32, (N_EXP, TOK), 0)
    oh0 = (jax.lax.broadcast_in_dim(a0r, (N_EXP, TOK), (0, 1)) == erow)
    oh1 = (jax.lax.broadcast_in_dim(a1r, (N_EXP, TOK), (0, 1)) == erow)
    oh0 = oh0.astype(jnp.int32)
    oh1 = oh1.astype(jnp.int32)
    oh = oh0 + oh1
    csum = oh
    shift = 1
    while shift < TOK:
        shifted = jnp.concatenate(
            [jnp.zeros((N_EXP, shift), jnp.int32), csum[:, :TOK - shift]],
            axis=1)
        csum = csum + shifted
        shift *= 2
    excl = csum - oh
    counts = csum[:, TOK - 1:TOK]
    tiles_e = (counts + TILE - 1) // TILE
    eids = lax.broadcasted_iota(jnp.int32, (N_EXP, 1), 0)
    rb = jax.lax.broadcast_in_dim(eids * SLOT, (N_EXP, TOK), (0, 1))
    pos0 = jnp.sum((excl + rb) * oh0, axis=0, keepdims=True)
    pos1 = jnp.sum((excl + rb) * oh1, axis=0, keepdims=True)
    p0_ref[...] = pos0
    p1_ref[...] = pos1
    nt_ref[...] = jnp.transpose(tiles_e)


def _run_router(fx, wr_pad, br_pad):
    return pl.pallas_call(
        _router_body,
        out_shape=(
            jax.ShapeDtypeStruct((1, TOK), jnp.int32),
            jax.ShapeDtypeStruct((1, TOK), jnp.int32),
            jax.ShapeDtypeStruct((1, TOK), jnp.float32),
            jax.ShapeDtypeStruct((1, TOK), jnp.float32),
            jax.ShapeDtypeStruct((1, N_EXP), jnp.int32),
        ),
        compiler_params=pltpu.CompilerParams(
            vmem_limit_bytes=64 * 1024 * 1024,
        ),
    )(fx, wr_pad, br_pad)


def _mlp_body(nt_ref, fx_ref, w1_ref, b1_ref, w2_ref, b2_ref,
              p0_ref, p1_ref, w0_ref, w1w_ref, ys_ref, xsc_ref, acc_ref):
    e = pl.program_id(0)
    hc = pl.program_id(1)
    ntile = nt_ref[e]

    def _masks(j):
        rows = lax.broadcasted_iota(jnp.int32, (TILE, TOK), 0) \
            + (e * SLOT + j * TILE)
        p0 = jax.lax.broadcast_in_dim(p0_ref[...], (TILE, TOK), (0, 1))
        p1 = jax.lax.broadcast_in_dim(p1_ref[...], (TILE, TOK), (0, 1))
        return p0 == rows, p1 == rows

    for j in range(MT):
        @pl.when(jnp.logical_and(j < ntile, hc == 0))
        def _(j=j):
            m0, m1 = _masks(j)
            onehot = (m0 | m1).astype(jnp.bfloat16)
            xs = jnp.dot(onehot, fx_ref[...],
                         preferred_element_type=jnp.float32).astype(jnp.bfloat16)
            xsc_ref[j] = xs
            h = jnp.dot(xs, w1_ref[0], preferred_element_type=jnp.float32)
            h = jnp.maximum(h + b1_ref[0], 0.0)
            acc_ref[j] = jnp.dot(h, w2_ref[0],
                                 preferred_element_type=jnp.float32)

        @pl.when(jnp.logical_and(j < ntile, hc == 1))
        def _(j=j):
            xs = xsc_ref[j]
            h = jnp.dot(xs, w1_ref[0], preferred_element_type=jnp.float32)
            h = jnp.maximum(h + b1_ref[0], 0.0)
            y = jnp.dot(h, w2_ref[0], preferred_element_type=jnp.float32)
            y = y + acc_ref[j] + b2_ref[0]
            m0, m1 = _masks(j)
            w0 = jax.lax.broadcast_in_dim(w0_ref[...], (TILE, TOK), (0, 1))
            w1w = jax.lax.broadcast_in_dim(w1w_ref[...], (TILE, TOK), (0, 1))
            zero = jnp.zeros((), jnp.float32)
            sw = jnp.sum(jnp.where(m0, w0, zero) + jnp.where(m1, w1w, zero),
                         axis=1, keepdims=True)
            ys_ref[pl.ds(j * TILE, TILE), :] = y * sw


def _run_mlp(nt, fxbf, W1, b1, W2, b2, p0, p1, w0, w1):
    const_spec = lambda shape: pl.BlockSpec(shape, lambda e, hc, nt: (0, 0))
    grid_spec = pltpu.PrefetchScalarGridSpec(
        num_scalar_prefetch=1,
        grid=(N_EXP, 2),
        in_specs=[
            pl.BlockSpec((TOK, D_MODEL), lambda e, hc, nt: (0, 0),
                         pipeline_mode=pl.Buffered(buffer_count=1)),
            pl.BlockSpec((1, D_MODEL, HHALF), lambda e, hc, nt: (e, 0, hc),
                         pipeline_mode=pl.Buffered(buffer_count=2)),
            pl.BlockSpec((1, 1, HHALF), lambda e, hc, nt: (e, 0, hc)),
            pl.BlockSpec((1, HHALF, D_MODEL), lambda e, hc, nt: (e, hc, 0),
                         pipeline_mode=pl.Buffered(buffer_count=2)),
            pl.BlockSpec((1, 1, D_MODEL), lambda e, hc, nt: (e, 0, 0)),
            const_spec((1, TOK)),
            const_spec((1, TOK)),
            const_spec((1, TOK)),
            const_spec((1, TOK)),
        ],
        out_specs=pl.BlockSpec((SLOT, D_MODEL), lambda e, hc, nt: (e, 0),
                               pipeline_mode=pl.Buffered(buffer_count=1)),
        scratch_shapes=[
            pltpu.VMEM((MT, TILE, D_MODEL), jnp.bfloat16),
            pltpu.VMEM((MT, TILE, D_MODEL), jnp.float32),
        ],
    )
    return pl.pallas_call(
        _mlp_body,
        grid_spec=grid_spec,
        out_shape=jax.ShapeDtypeStruct((YS_ROWS, D_MODEL), jnp.float32),
        compiler_params=pltpu.CompilerParams(
            dimension_semantics=("arbitrary", "arbitrary"),
            vmem_limit_bytes=120 * 1024 * 1024,
        ),
    )(nt, fxbf, W1, b1, W2, b2, p0, p1, w0, w1)


_C_TOK = TOK // NW
_C_CH = 16
_C_NCH = _C_TOK // _C_CH


@functools.partial(
    pl.kernel,
    mesh=_MESH,
    out_type=jax.ShapeDtypeStruct((TOK, D_MODEL), jnp.float32),
    scratch_types=[
        pltpu.VMEM((_C_TOK,), jnp.int32),
        pltpu.VMEM((_C_TOK,), jnp.int32),
        pltpu.VMEM((_C_CH, D_MODEL), jnp.float32),
        pltpu.VMEM((_C_CH, D_MODEL), jnp.float32),
        pltpu.VMEM((_C_CH, D_MODEL), jnp.float32),
        pltpu.VMEM((_C_CH, D_MODEL), jnp.float32),
        pltpu.SemaphoreType.DMA,
        pltpu.SemaphoreType.DMA,
        pltpu.SemaphoreType.DMA,
        pltpu.SemaphoreType.DMA,
    ],
)
def _sc_combine(ys_hbm, p0_hbm, p1_hbm, out_hbm,
                i0_v, i1_v, a0_v, b0_v, a1_v, b1_v, sa0, sb0, sa1, sb1):
    wid = lax.axis_index("s") * _NC + lax.axis_index("c")
    base = wid * _C_TOK
    pltpu.sync_copy(p0_hbm.at[pl.ds(base, _C_TOK)], i0_v)
    pltpu.sync_copy(p1_hbm.at[pl.ds(base, _C_TOK)], i1_v)
    abufs = (a0_v, a1_v)
    bbufs = (b0_v, b1_v)
    sems = ((sa0, sb0), (sa1, sb1))
    cps = [None, None]

    def _fire(c, p):
        sl = pl.ds(c * _C_CH, _C_CH)
        cpa = pltpu.async_copy(ys_hbm.at[i0_v.at[sl]], abufs[p], sems[p][0])
        cpb = pltpu.async_copy(ys_hbm.at[i1_v.at[sl]], bbufs[p], sems[p][1])
        return (cpa, cpb)

    cps[0] = _fire(0, 0)
    for c in range(_C_NCH):
        p = c % 2
        if c + 1 < _C_NCH:
            cps[(c + 1) % 2] = _fire(c + 1, (c + 1) % 2)
        cps[p][0].wait()
        cps[p][1].wait()
        a_v, b_v = abufs[p], bbufs[p]

        def _row(r, _):
            for u in range(D_MODEL // 16):
                sl = pl.ds(u * 16, 16)
                a_v[r, sl] = a_v[r, sl] + b_v[r, sl]
            return 0

        lax.fori_loop(0, _C_CH, _row, 0)
        pltpu.sync_copy(a_v, out_hbm.at[pl.ds(base + c * _C_CH, _C_CH)])


def kernel(x, Wr, br, W1, b1, W2, b2):
    B, S, D = x.shape
    fx = x.reshape(B * S, D)

    wr_pad = jnp.zeros((D_MODEL, 128), jnp.float32).at[:, :N_EXP].set(Wr)
    br_pad = jnp.zeros((1, 128), jnp.float32).at[0, :N_EXP].set(br)
    p0, p1, w0, w1, nt = _run_router(fx, wr_pad, br_pad)

    ys = _run_mlp(nt[0], fx.astype(jnp.bfloat16), W1,
                  b1.reshape(N_EXP, 1, HID), W2,
                  b2.reshape(N_EXP, 1, D_MODEL), p0, p1, w0, w1)

    out = _sc_combine(ys, p0.reshape(TOK), p1.reshape(TOK))
    return out.reshape(B, S, D)


# speedup vs baseline: 2.6689x; 2.6689x over previous
import functools

import jax
import jax.numpy as jnp
from jax import lax
from jax.experimental import pallas as pl
from jax.experimental.pallas import tpu as pltpu
from jax.experimental.pallas import tpu_sc as plsc

D_MODEL = 1024
HID = 4096
N_EXP = 8
TOPK = 2
TOK = 2048
TILE = 256
G = (TOK * TOPK) // TILE + N_EXP
GP = G * TILE

_SC_INFO = plsc.get_sparse_core_info()
_NC = _SC_INFO.num_cores
_NS = _SC_INFO.num_subcores
NW = _NC * _NS

_MESH = plsc.VectorSubcoreMesh(core_axis_name="c", subcore_axis_name="s")


def _router_body(x_ref, wr_ref, br_ref,
                 p0_ref, p1_ref, w0_ref, w1_ref, te_ref, tv_ref):
    logits = jnp.dot(x_ref[...], wr_ref[...], preferred_element_type=jnp.float32)
    logits = logits + br_ref[...]
    cols = lax.broadcasted_iota(jnp.int32, logits.shape, 1)
    neg = jnp.float32(-jnp.inf)
    l0 = jnp.where(cols < N_EXP, logits, neg)
    v0 = jnp.max(l0, axis=1, keepdims=True)
    a0 = jnp.min(jnp.where(l0 == v0, cols, N_EXP), axis=1, keepdims=True)
    l1 = jnp.where(cols == a0, neg, l0)
    v1 = jnp.max(l1, axis=1, keepdims=True)
    a1 = jnp.min(jnp.where(l1 == v1, cols, N_EXP), axis=1, keepdims=True)
    t = jnp.exp(v1 - v0)
    w0 = 1.0 / (1.0 + t)
    w0_ref[...] = jnp.transpose(w0)
    w1_ref[...] = jnp.transpose(1.0 - w0)

    a0r = jnp.transpose(a0)
    a1r = jnp.transpose(a1)
    erow = lax.broadcasted_iota(jnp.int32, (N_EXP, TOK), 0)
    oh0 = (jax.lax.broadcast_in_dim(a0r, (N_EXP, TOK), (0, 1)) == erow)
    oh1 = (jax.lax.broadcast_in_dim(a1r, (N_EXP, TOK), (0, 1)) == erow)
    oh0 = oh0.astype(jnp.int32)
    oh1 = oh1.astype(jnp.int32)
    oh = oh0 + oh1
    csum = oh
    shift = 1
    while shift < TOK:
        shifted = jnp.concatenate(
            [jnp.zeros((N_EXP, shift), jnp.int32), csum[:, :TOK - shift]],
            axis=1)
        csum = csum + shifted
        shift *= 2
    excl = csum - oh
    counts = csum[:, TOK - 1:TOK]
    tiles_e = (counts + TILE - 1) // TILE
    ct = tiles_e
    shift = 1
    while shift < N_EXP:
        shifted = jnp.concatenate(
            [jnp.zeros((shift, 1), jnp.int32), ct[:N_EXP - shift, :]], axis=0)
        ct = ct + shifted
        shift *= 2
    row_base = (ct - tiles_e) * TILE
    rb = jax.lax.broadcast_in_dim(row_base, (N_EXP, TOK), (0, 1))
    pos0 = jnp.sum((excl + rb) * oh0, axis=0, keepdims=True)
    pos1 = jnp.sum((excl + rb) * oh1, axis=0, keepdims=True)
    p0_ref[...] = pos0
    p1_ref[...] = pos1

    tid = lax.broadcasted_iota(jnp.int32, (1, G), 1)
    ctb = jax.lax.broadcast_in_dim(ct, (N_EXP, G), (0, 1))
    te = jnp.sum((ctb <= tid).astype(jnp.int32), axis=0, keepdims=True)
    total = ct[N_EXP - 1:N_EXP, :]
    tv = (tid < jax.lax.broadcast_in_dim(total, (1, G), (0, 1)))
    eids = lax.broadcasted_iota(jnp.int32, (N_EXP, 1), 0)
    last_e = jnp.max(jnp.where(tiles_e > 0, eids, 0))
    te = jnp.where(tv, jnp.minimum(te, N_EXP - 1), last_e)
    te_ref[...] = te
    tv_ref[...] = tv.astype(jnp.int32)


def _run_router(fx, wr_pad, br_pad):
    return pl.pallas_call(
        _router_body,
        out_shape=(
            jax.ShapeDtypeStruct((1, TOK), jnp.int32),
            jax.ShapeDtypeStruct((1, TOK), jnp.int32),
            jax.ShapeDtypeStruct((1, TOK), jnp.float32),
            jax.ShapeDtypeStruct((1, TOK), jnp.float32),
            jax.ShapeDtypeStruct((1, G), jnp.int32),
            jax.ShapeDtypeStruct((1, G), jnp.int32),
        ),
        compiler_params=pltpu.CompilerParams(
            vmem_limit_bytes=64 * 1024 * 1024,
        ),
    )(fx, wr_pad, br_pad)


def _mlp_body(te_ref, tv_ref, fx_ref, w1_ref, b1_ref, w2_ref, b2_ref,
              p0_ref, p1_ref, w0_ref, w1w_ref, ys_ref):
    i = pl.program_id(0)

    @pl.when(tv_ref[i] != 0)
    def _():
        rows = lax.broadcasted_iota(jnp.int32, (TILE, TOK), 0) + i * TILE
        p0 = jax.lax.broadcast_in_dim(p0_ref[...], (TILE, TOK), (0, 1))
        p1 = jax.lax.broadcast_in_dim(p1_ref[...], (TILE, TOK), (0, 1))
        m0 = p0 == rows
        m1 = p1 == rows
        onehot = (m0 | m1).astype(jnp.float32)
        xs = jnp.dot(onehot, fx_ref[...], preferred_element_type=jnp.float32)
        h = jnp.dot(xs, w1_ref[0], preferred_element_type=jnp.float32)
        h = jnp.maximum(h + b1_ref[0], 0.0)
        y = jnp.dot(h, w2_ref[0], preferred_element_type=jnp.float32)
        y = y + b2_ref[0]
        w0 = jax.lax.broadcast_in_dim(w0_ref[...], (TILE, TOK), (0, 1))
        w1w = jax.lax.broadcast_in_dim(w1w_ref[...], (TILE, TOK), (0, 1))
        zero = jnp.zeros((), jnp.float32)
        sw = jnp.sum(jnp.where(m0, w0, zero) + jnp.where(m1, w1w, zero),
                     axis=1, keepdims=True)
        ys_ref[...] = y * sw


def _run_mlp(te, tv, fxbf, W1, b1, W2, b2, p0, p1, w0, w1):
    const_spec = lambda shape: pl.BlockSpec(shape, lambda i, te, tv: (0, 0))
    grid_spec = pltpu.PrefetchScalarGridSpec(
        num_scalar_prefetch=2,
        grid=(G,),
        in_specs=[
            pl.BlockSpec((TOK, D_MODEL), lambda i, te, tv: (0, 0),
                         pipeline_mode=pl.Buffered(buffer_count=1)),
            pl.BlockSpec((1, D_MODEL, HID), lambda i, te, tv: (te[i], 0, 0),
                         pipeline_mode=pl.Buffered(buffer_count=2)),
            pl.BlockSpec((1, 1, HID), lambda i, te, tv: (te[i], 0, 0)),
            pl.BlockSpec((1, HID, D_MODEL), lambda i, te, tv: (te[i], 0, 0),
                         pipeline_mode=pl.Buffered(buffer_count=1)),
            pl.BlockSpec((1, 1, D_MODEL), lambda i, te, tv: (te[i], 0, 0)),
            const_spec((1, TOK)),
            const_spec((1, TOK)),
            const_spec((1, TOK)),
            const_spec((1, TOK)),
        ],
        out_specs=pl.BlockSpec((TILE, D_MODEL), lambda i, te, tv: (i, 0)),
    )
    return pl.pallas_call(
        _mlp_body,
        grid_spec=grid_spec,
        out_shape=jax.ShapeDtypeStruct((GP, D_MODEL), jnp.float32),
        compiler_params=pltpu.CompilerParams(
            dimension_semantics=("arbitrary",),
            vmem_limit_bytes=120 * 1024 * 1024,
        ),
    )(te, tv, fxbf, W1, b1, W2, b2, p0, p1, w0, w1)


_C_TOK = TOK // NW
_C_CH = 16
_C_NCH = _C_TOK // _C_CH


@functools.partial(
    pl.kernel,
    mesh=_MESH,
    out_type=jax.ShapeDtypeStruct((TOK, D_MODEL), jnp.float32),
    scratch_types=[
        pltpu.VMEM((_C_TOK,), jnp.int32),
        pltpu.VMEM((_C_TOK,), jnp.int32),
        pltpu.VMEM((_C_CH, D_MODEL), jnp.float32),
        pltpu.VMEM((_C_CH, D_MODEL), jnp.float32),
        pltpu.VMEM((_C_CH, D_MODEL), jnp.float32),
        pltpu.VMEM((_C_CH, D_MODEL), jnp.float32),
        pltpu.SemaphoreType.DMA,
        pltpu.SemaphoreType.DMA,
        pltpu.SemaphoreType.DMA,
        pltpu.SemaphoreType.DMA,
    ],
)
def _sc_combine(ys_hbm, p0_hbm, p1_hbm, out_hbm,
                i0_v, i1_v, a0_v, b0_v, a1_v, b1_v, sa0, sb0, sa1, sb1):
    wid = lax.axis_index("s") * _NC + lax.axis_index("c")
    base = wid * _C_TOK
    pltpu.sync_copy(p0_hbm.at[pl.ds(base, _C_TOK)], i0_v)
    pltpu.sync_copy(p1_hbm.at[pl.ds(base, _C_TOK)], i1_v)
    abufs = (a0_v, a1_v)
    bbufs = (b0_v, b1_v)
    sems = ((sa0, sb0), (sa1, sb1))
    cps = [None, None]

    def _fire(c, p):
        sl = pl.ds(c * _C_CH, _C_CH)
        cpa = pltpu.async_copy(ys_hbm.at[i0_v.at[sl]], abufs[p], sems[p][0])
        cpb = pltpu.async_copy(ys_hbm.at[i1_v.at[sl]], bbufs[p], sems[p][1])
        return (cpa, cpb)

    cps[0] = _fire(0, 0)
    for c in range(_C_NCH):
        p = c % 2
        if c + 1 < _C_NCH:
            cps[(c + 1) % 2] = _fire(c + 1, (c + 1) % 2)
        cps[p][0].wait()
        cps[p][1].wait()
        a_v, b_v = abufs[p], bbufs[p]

        def _row(r, _):
            for u in range(D_MODEL // 16):
                sl = pl.ds(u * 16, 16)
                a_v[r, sl] = a_v[r, sl] + b_v[r, sl]
            return 0

        lax.fori_loop(0, _C_CH, _row, 0)
        pltpu.sync_copy(a_v, out_hbm.at[pl.ds(base + c * _C_CH, _C_CH)])


def kernel(x, Wr, br, W1, b1, W2, b2):
    B, S, D = x.shape
    fx = x.reshape(B * S, D)

    wr_pad = jnp.zeros((D_MODEL, 128), jnp.float32).at[:, :N_EXP].set(Wr)
    br_pad = jnp.zeros((1, 128), jnp.float32).at[0, :N_EXP].set(br)
    p0, p1, w0, w1, te, tv = _run_router(fx, wr_pad, br_pad)

    ys = _run_mlp(te[0], tv[0], fx, W1,
                  b1.reshape(N_EXP, 1, HID), W2,
                  b2.reshape(N_EXP, 1, D_MODEL), p0, p1, w0, w1)

    out = _sc_combine(ys, p0.reshape(TOK), p1.reshape(TOK))
    return out.reshape(B, S, D)
